# baseline (speedup 1.0000x reference)
import functools

import jax
import jax.numpy as jnp
from jax import lax
from jax.experimental import pallas as pl
from jax.experimental.pallas import tpu as pltpu
from jax.experimental.pallas import tpu_sc as plsc

B = 4096
L = 200
D = 64
NC = 2
NS = 16
NW = NC * NS
BPW = B // NW
CPB = 2
CL = L // CPB
NCH = D // 16
NG = CL // 16
TAIL = CL - 16 * NG
OPAD = 16 * (NG + 1)

_mesh = plsc.VectorSubcoreMesh(core_axis_name="c", subcore_axis_name="s")


@functools.partial(
    pl.kernel,
    mesh=_mesh,
    out_type=jax.ShapeDtypeStruct((B // 2, 2 * D), jnp.float32),
    scratch_types=[
        pltpu.VMEM((BPW * CPB, CL), jnp.int32),
        pltpu.VMEM((BPW * CPB, OPAD), jnp.int32),
        pltpu.VMEM((CL, 2 * D), jnp.float32),
        pltpu.VMEM((CL, 2 * D), jnp.float32),
        pltpu.VMEM((CL, 2 * D), jnp.float32),
        pltpu.VMEM((CL, 2 * D), jnp.float32),
        pltpu.VMEM((BPW // 2, 2 * D), jnp.float32),
        pltpu.SemaphoreType.DMA,
        pltpu.SemaphoreType.DMA,
        pltpu.SemaphoreType.DMA,
        pltpu.SemaphoreType.DMA,
    ],
)
def _emb_pool(xp_hbm, xo_hbm, table_hbm, out_hbm, idx_v, off_v,
              ra0, ra1, rb0, rb1, out_v, sa0, sa1, sb0, sb1):
    wid = lax.axis_index("s") * NC + lax.axis_index("c")
    base = wid * BPW * CPB
    pltpu.sync_copy(xp_hbm.at[pl.ds(base, BPW * CPB)], idx_v)
    pltpu.sync_copy(xo_hbm.at[pl.ds(base, BPW * CPB)], off_v)

    pair_a = ((ra0, sa0), (ra1, sa1))
    pair_b = ((rb0, sb0), (rb1, sb1))

    def descs(elt, pair):
        return [
            pltpu.make_async_copy(table_hbm.at[idx_v.at[CPB * elt + k]], buf, sem)
            for k, (buf, sem) in enumerate(pair)
        ]

    def start(elt, pair):
        for d in descs(elt, pair):
            d.start()

    def wait(elt, pair):
        for d in descs(elt, pair):
            d.wait()

    def chunk_sums(off_row, buf):
        def group(g, nrows, accs):
            q16 = off_v[off_row, pl.ds(16 * g, 16)]
            new = list(accs)
            for k in range(nrows):
                o = q16[k]
                for c in range(NCH):
                    new[c] = new[c] + buf[16 * g + k, pl.ds(o + c * 16, 16)]
            return tuple(new)

        accs = lax.fori_loop(
            0, NG, lambda g, a: group(g, 16, a),
            tuple(jnp.zeros((16,), jnp.float32) for _ in range(NCH)),
        )
        return group(NG, TAIL, accs)

    def accumulate(elt, pair, row, half):
        sums = [
            chunk_sums(CPB * elt + k, buf) for k, (buf, _) in enumerate(pair)
        ]
        for c in range(NCH):
            out_v[row, pl.ds(half * D + c * 16, 16)] = (
                (sums[0][c] + sums[1][c]) * (1.0 / L)
            )

    start(0, pair_a)

    def outer(i, carry):
        b0 = 2 * i
        start(b0 + 1, pair_b)
        wait(b0, pair_a)
        accumulate(b0, pair_a, i, 0)
        start(jnp.minimum(b0 + 2, BPW - 1), pair_a)
        wait(b0 + 1, pair_b)
        accumulate(b0 + 1, pair_b, i, 1)
        return carry

    lax.fori_loop(0, BPW // 2, outer, 0)
    wait(BPW - 1, pair_a)
    pltpu.sync_copy(out_v, out_hbm.at[pl.ds(wid * (BPW // 2), BPW // 2)])


def kernel(x, table):
    xi = x.astype(jnp.int32)
    xp = (xi >> 1).reshape(B * CPB, CL)
    xo = jnp.pad(((xi & 1) << 6).reshape(B * CPB, CL),
                 ((0, 0), (0, OPAD - CL)))
    t2 = table.reshape(table.shape[0] // 2, 2 * D)
    return _emb_pool(xp, xo, t2).reshape(B, D)


# speedup vs baseline: 2.5814x; 2.5814x over previous
import functools

import jax
import jax.numpy as jnp
from jax import lax
from jax.experimental import pallas as pl
from jax.experimental.pallas import tpu as pltpu
from jax.experimental.pallas import tpu_sc as plsc

B = 4096
L = 200
D = 64
NC = 2
NS = 16
NW = NC * NS
BPW = B // NW
NCH = D // 16
NG = L // 16
TAIL = L - 16 * NG
UN = 4

_mesh = plsc.VectorSubcoreMesh(core_axis_name="c", subcore_axis_name="s")


@functools.partial(
    pl.kernel,
    mesh=_mesh,
    out_type=jax.ShapeDtypeStruct((B // 2, 2 * D), jnp.float32),
    scratch_types=[
        pltpu.VMEM((BPW, L), jnp.int32),
        pltpu.VMEM((L, D), jnp.float32),
        pltpu.VMEM((L, D), jnp.float32),
        pltpu.VMEM((BPW // 2, 2 * D), jnp.float32),
        pltpu.SemaphoreType.DMA,
        pltpu.SemaphoreType.DMA,
    ],
)
def _emb_pool(x_hbm, table_hbm, out_hbm, idx_v, rows_a, rows_b, out_v,
              sem_a, sem_b):
    wid = lax.axis_index("s") * NC + lax.axis_index("c")
    pltpu.sync_copy(x_hbm.at[pl.ds(wid * BPW, BPW)], idx_v)

    def issue(elt, buf, sem):
        def issue_group(g, carry):
            base = 16 * g
            q16 = idx_v[elt, pl.ds(base, 16)]
            for k in range(16):
                pltpu.make_async_copy(
                    table_hbm.at[pl.ds(q16[k], 1)],
                    buf.at[pl.ds(base + k, 1)],
                    sem,
                ).start()
            return carry

        lax.fori_loop(0, NG, issue_group, 0)
        q16 = idx_v[elt, pl.ds(L - 16, 16)]
        for k in range(16 - TAIL, 16):
            pltpu.make_async_copy(
                table_hbm.at[pl.ds(q16[k], 1)],
                buf.at[pl.ds(L - 16 + k, 1)],
                sem,
            ).start()

    def drain(buf, sem):
        pltpu.make_async_copy(table_hbm.at[pl.ds(0, L)], buf, sem).wait()

    def accumulate(buf, row, half):
        def acc_body(j, accs):
            r = j * UN
            new = list(accs)
            for k in range(UN):
                for c in range(NCH):
                    new[c] = new[c] + buf[r + k, pl.ds(c * 16, 16)]
            return tuple(new)

        accs = lax.fori_loop(
            0, L // UN, acc_body,
            tuple(jnp.zeros((16,), jnp.float32) for _ in range(NCH)),
        )
        for c in range(NCH):
            out_v[row, pl.ds(half * D + c * 16, 16)] = accs[c] * (1.0 / L)

    issue(0, rows_a, sem_a)

    def outer(i, carry):
        b0 = 2 * i
        issue(b0 + 1, rows_b, sem_b)
        drain(rows_a, sem_a)
        accumulate(rows_a, i, 0)
        issue(jnp.minimum(b0 + 2, BPW - 1), rows_a, sem_a)
        drain(rows_b, sem_b)
        accumulate(rows_b, i, 1)
        return carry

    lax.fori_loop(0, BPW // 2, outer, 0)
    drain(rows_a, sem_a)
    pltpu.sync_copy(out_v, out_hbm.at[pl.ds(wid * (BPW // 2), BPW // 2)])


def kernel(x, table):
    return _emb_pool(x.astype(jnp.int32), table).reshape(B, D)
